# baseline (speedup 1.0000x reference)
import functools

import jax
import jax.numpy as jnp
from jax import lax
from jax.experimental import pallas as pl
from jax.experimental.pallas import tpu as pltpu
from jax.experimental.pallas import tpu_sc as plsc

MODEL = 128
NHEAD = 4
NCELLS = 512
VOCAB = 1000
MEM = 64
B, T = 8, 128
IFACE_PAD = 512


def _make_sc_gather():
    info = plsc.get_sparse_core_info()
    nc, ns = info.num_cores, info.num_subcores
    nw = nc * ns
    n_idx = T * B
    per_w = n_idx // nw
    mesh = plsc.VectorSubcoreMesh(core_axis_name="c", subcore_axis_name="s")

    @functools.partial(
        pl.kernel,
        mesh=mesh,
        out_type=jax.ShapeDtypeStruct((n_idx, MODEL), jnp.float32),
        scratch_types=[
            pltpu.VMEM((per_w,), jnp.int32),
            pltpu.VMEM((per_w, MODEL), jnp.float32),
            pltpu.SemaphoreType.DMA,
        ],
    )
    def gather(table_hbm, idx_hbm, out_hbm, idx_v, rows_v, sem):
        wid = lax.axis_index("s") * nc + lax.axis_index("c")
        base = wid * per_w
        pltpu.sync_copy(idx_hbm.at[pl.ds(base, per_w)], idx_v)
        pltpu.async_copy(table_hbm.at[idx_v], rows_v, sem).wait()
        pltpu.sync_copy(rows_v, out_hbm.at[pl.ds(base, per_w)])

    return gather


def _dot(a, b, ca, cb):
    return lax.dot_general(
        a, b, (((ca,), (cb,)), ((), ())), preferred_element_type=jnp.float32
    )


def _bdot(a, b, ca, cb):
    return lax.dot_general(
        a, b, (((ca,), (cb,)), ((0,), (0,))), preferred_element_type=jnp.float32
    )


def _softplus(x):
    return jnp.maximum(x, 0.0) + jnp.log(1.0 + jnp.exp(-jnp.abs(x)))


def _dnc_body(
    xs_ref, wall_ref, bl_ref, wif_ref, bif_ref,
    wouth_ref, woutr_ref, wfc_ref, bfc_ref, out_ref,
    m8, hB, cB, rB, nrm, outs_h, outs_r, wwS, evS, wvS,
):
    m8[...] = jnp.zeros_like(m8)
    hB[...] = jnp.zeros_like(hB)
    cB[...] = jnp.zeros_like(cB)
    rB[...] = jnp.zeros_like(rB)
    nrm[...] = jnp.zeros_like(nrm)
    wwS[...] = jnp.zeros_like(wwS)
    evS[...] = jnp.zeros_like(evS)
    wvS[...] = jnp.zeros_like(wvS)

    wall = wall_ref[...]
    bl = bl_ref[...]
    wif = wif_ref[...]
    bif = bif_ref[...]

    def step(t, carry):
        wwp = wwS[...][:, None, :]
        m = m8[...] * (1.0 - wwp * evS[...][:, :, None]) + wwp * wvS[...][:, :, None]
        m8[...] = m
        msq = m * m

        x_t = xs_ref[t]
        catv = jnp.concatenate([x_t, rB[...], hB[...]], axis=1)
        gates = _dot(catv, wall, 1, 1) + bl
        ig = jax.nn.sigmoid(gates[:, 0:MODEL])
        fg = jax.nn.sigmoid(gates[:, MODEL : 2 * MODEL])
        gg = jnp.tanh(gates[:, 2 * MODEL : 3 * MODEL])
        og = jax.nn.sigmoid(gates[:, 3 * MODEL : 4 * MODEL])
        c_new = fg * cB[...] + ig * gg
        h_new = og * jnp.tanh(c_new)
        cB[...] = c_new
        hB[...] = h_new
        outs_h[t] = h_new

        iface = _dot(h_new, wif, 1, 1) + bif
        wk = iface[:, NHEAD * MEM : NHEAD * MEM + MEM]
        wv = iface[:, NHEAD * MEM + MEM : NHEAD * MEM + 2 * MEM]
        ev = jax.nn.sigmoid(iface[:, NHEAD * MEM + 2 * MEM : NHEAD * MEM + 3 * MEM])
        betas = _softplus(iface[:, NHEAD * MEM + 3 * MEM : NHEAD * MEM + 3 * MEM + NHEAD + 1]) + 1.0
        rbeta = betas[:, 0:NHEAD]
        wbeta = betas[:, NHEAD : NHEAD + 1]

        rk_all = iface[:, 0 : NHEAD * MEM].reshape(B, NHEAD, MEM)
        wknorm = jnp.sqrt(jnp.sum(wk * wk, axis=1, keepdims=True))
        rknorm = jnp.sqrt(jnp.sum(rk_all * rk_all, axis=2, keepdims=True))

        evrk = rk_all * ev[:, None, :]
        evwv = ev * wv
        kmat = jnp.concatenate(
            [wk[:, None, :], rk_all, evrk, wv[:, None, :], evwv[:, None, :]],
            axis=1,
        )
        qmat = jnp.concatenate(
            [jnp.ones((B, 1, MEM), jnp.float32), ev[:, None, :], (ev * ev)[:, None, :]],
            axis=1,
        )
        sims = _bdot(kmat, m, 2, 1)
        sq = _bdot(qmat, msq, 2, 1)
        s_rv = jnp.sum(rk_all * wv[:, None, :], axis=2, keepdims=True)
        s_vv = jnp.sum(wv * wv, axis=1)[:, None, None]

        simw = sims[:, 0] / (nrm[...] + 1e-6) * (wbeta / (wknorm + 1e-6))
        mx = jnp.max(simw, axis=-1, keepdims=True)
        e = jnp.exp(simw - mx)
        ww = e / jnp.sum(e, axis=-1, keepdims=True)

        ww3 = ww[:, None, :]
        simr_raw = sims[:, 1 : 1 + NHEAD] - ww3 * sims[:, 1 + NHEAD : 1 + 2 * NHEAD] + ww3 * s_rv
        nrm2_new = (
            sq[:, 0:1]
            - 2.0 * ww3 * sq[:, 1:2]
            + (ww3 * ww3) * sq[:, 2:3]
            + 2.0 * ww3 * sims[:, 9:10]
            - 2.0 * (ww3 * ww3) * sims[:, 10:11]
            + (ww3 * ww3) * s_vv
        )
        nrm_new = jnp.sqrt(jnp.maximum(nrm2_new, 0.0))
        nrm[...] = nrm_new[:, 0]

        simr = simr_raw / (nrm_new + 1e-6) * (rbeta[:, :, None] / (rknorm + 1e-6))
        mxr = jnp.max(simr, axis=-1, keepdims=True)
        er = jnp.exp(simr - mxr)
        wr = er / jnp.sum(er, axis=-1, keepdims=True)

        wrw = wr * ww3
        wr2 = jnp.concatenate([wr, wrw], axis=1)
        rsum = _bdot(wr2, m, 2, 2)
        srw = jnp.sum(wrw, axis=2, keepdims=True)
        reads = (
            rsum[:, 0:NHEAD]
            - rsum[:, NHEAD : 2 * NHEAD] * ev[:, None, :]
            + srw * wv[:, None, :]
        )
        r_new = reads.reshape(B, NHEAD * MEM)
        rB[...] = r_new
        outs_r[t] = r_new
        wwS[...] = ww
        evS[...] = ev
        wvS[...] = wv
        return carry

    lax.fori_loop(0, T, step, 0)

    wouth = wouth_ref[...]
    woutr = woutr_ref[...]
    wfc = wfc_ref[...]
    bfc = bfc_ref[...]
    for b in range(B):
        src_b = (
            _dot(outs_h[:, b, :], wouth, 1, 1)
            + _dot(outs_r[:, b, :], woutr, 1, 1)
        )
        out_ref[b] = _dot(wfc, src_b, 1, 1) + bfc


def _recurrence(xs, wall, bl, wifp, bifp, wouth, woutr, wfc, bfc):
    return pl.pallas_call(
        _dnc_body,
        out_shape=jax.ShapeDtypeStruct((B, VOCAB, T), jnp.float32),
        scratch_shapes=[
            pltpu.VMEM((B, MEM, NCELLS), jnp.float32),
            pltpu.VMEM((B, MODEL), jnp.float32),
            pltpu.VMEM((B, MODEL), jnp.float32),
            pltpu.VMEM((B, NHEAD * MEM), jnp.float32),
            pltpu.VMEM((B, NCELLS), jnp.float32),
            pltpu.VMEM((T, B, MODEL), jnp.float32),
            pltpu.VMEM((T, B, NHEAD * MEM), jnp.float32),
            pltpu.VMEM((B, NCELLS), jnp.float32),
            pltpu.VMEM((B, MEM), jnp.float32),
            pltpu.VMEM((B, MEM), jnp.float32),
        ],
    )(xs, wall, bl, wifp, bifp, wouth, woutr, wfc, bfc)


def kernel(input, emb, W_ih, W_hh, b_lstm, W_if, b_if, W_out, b_out, W_fc, b_fc):
    idx = jnp.swapaxes(input, 0, 1).reshape(T * B).astype(jnp.int32)
    rows = _make_sc_gather()(emb, idx)
    xs = rows.reshape(T, B, MODEL)

    wall = jnp.concatenate([W_ih, W_hh], axis=1)
    bl = b_lstm.reshape(1, -1)
    iface_dim = W_if.shape[0]
    wifp = jnp.zeros((IFACE_PAD, MODEL), jnp.float32).at[:iface_dim].set(W_if)
    bifp = jnp.zeros((1, IFACE_PAD), jnp.float32).at[0, :iface_dim].set(b_if)
    wouth = W_out[:, :MODEL]
    woutr = W_out[:, MODEL:]
    bfc = (W_fc @ b_out + b_fc).reshape(-1, 1)

    return _recurrence(xs, wall, bl, wifp, bifp, wouth, woutr, W_fc, bfc)


# speedup vs baseline: 1.1229x; 1.1229x over previous
import functools

import jax
import jax.numpy as jnp
from jax import lax
from jax.experimental import pallas as pl
from jax.experimental.pallas import tpu as pltpu
from jax.experimental.pallas import tpu_sc as plsc

MODEL = 128
NHEAD = 4
NCELLS = 512
VOCAB = 1000
MEM = 64
B, T = 8, 128
IFACE_PAD = 512


def _make_sc_gather():
    info = plsc.get_sparse_core_info()
    nc, ns = info.num_cores, info.num_subcores
    nw = nc * ns
    n_idx = T * B
    per_w = n_idx // nw
    mesh = plsc.VectorSubcoreMesh(core_axis_name="c", subcore_axis_name="s")

    @functools.partial(
        pl.kernel,
        mesh=mesh,
        out_type=jax.ShapeDtypeStruct((n_idx, MODEL), jnp.float32),
        scratch_types=[
            pltpu.VMEM((per_w,), jnp.int32),
            pltpu.VMEM((per_w, MODEL), jnp.float32),
            pltpu.SemaphoreType.DMA,
        ],
    )
    def gather(table_hbm, idx_hbm, out_hbm, idx_v, rows_v, sem):
        wid = lax.axis_index("s") * nc + lax.axis_index("c")
        base = wid * per_w
        pltpu.sync_copy(idx_hbm.at[pl.ds(base, per_w)], idx_v)
        pltpu.async_copy(table_hbm.at[idx_v], rows_v, sem).wait()
        pltpu.sync_copy(rows_v, out_hbm.at[pl.ds(base, per_w)])

    return gather


def _dot(a, b, ca, cb):
    return lax.dot_general(
        a, b, (((ca,), (cb,)), ((), ())), preferred_element_type=jnp.float32
    )


def _bdot(a, b, ca, cb):
    return lax.dot_general(
        a, b, (((ca,), (cb,)), ((0,), (0,))), preferred_element_type=jnp.float32
    )


def _softplus(x):
    return jnp.maximum(x, 0.0) + jnp.log(1.0 + jnp.exp(-jnp.abs(x)))


def _dnc_body(
    xs_ref, wall_ref, bl_ref, wif_ref, bif_ref,
    wouth_ref, woutr_ref, wfc_ref, bfc_ref, out_ref,
    m8, hB, cB, rB, nrm, outs_h, outs_r, wwS, evS, wvS,
):
    m8[...] = jnp.zeros_like(m8)
    hB[...] = jnp.zeros_like(hB)
    cB[...] = jnp.zeros_like(cB)
    rB[...] = jnp.zeros_like(rB)
    nrm[...] = jnp.zeros_like(nrm)
    wwS[...] = jnp.zeros_like(wwS)
    evS[...] = jnp.zeros_like(evS)
    wvS[...] = jnp.zeros_like(wvS)

    wall = wall_ref[...]
    bl = bl_ref[...]
    wif = wif_ref[...]
    bif = bif_ref[...]

    def step(t, carry):
        wwp = wwS[...][:, None, :]
        m = m8[...] * (1.0 - wwp * evS[...][:, :, None]) + wwp * wvS[...][:, :, None]
        m8[...] = m
        msq = m * m

        x_t = xs_ref[t]
        catv = jnp.concatenate([x_t, rB[...], hB[...]], axis=1)
        gates = _dot(catv, wall, 1, 1) + bl
        ig = jax.nn.sigmoid(gates[:, 0:MODEL])
        fg = jax.nn.sigmoid(gates[:, MODEL : 2 * MODEL])
        gg = jnp.tanh(gates[:, 2 * MODEL : 3 * MODEL])
        og = jax.nn.sigmoid(gates[:, 3 * MODEL : 4 * MODEL])
        c_new = fg * cB[...] + ig * gg
        h_new = og * jnp.tanh(c_new)
        cB[...] = c_new
        hB[...] = h_new
        outs_h[t] = h_new

        iface = _dot(h_new, wif, 1, 1) + bif
        wk = iface[:, NHEAD * MEM : NHEAD * MEM + MEM]
        wv = iface[:, NHEAD * MEM + MEM : NHEAD * MEM + 2 * MEM]
        ev = jax.nn.sigmoid(iface[:, NHEAD * MEM + 2 * MEM : NHEAD * MEM + 3 * MEM])
        betas = _softplus(iface[:, NHEAD * MEM + 3 * MEM : NHEAD * MEM + 3 * MEM + NHEAD + 1]) + 1.0
        rbeta = betas[:, 0:NHEAD]
        wbeta = betas[:, NHEAD : NHEAD + 1]

        rk_all = iface[:, 0 : NHEAD * MEM].reshape(B, NHEAD, MEM)
        wknorm = jnp.sqrt(jnp.sum(wk * wk, axis=1, keepdims=True))
        rknorm = jnp.sqrt(jnp.sum(rk_all * rk_all, axis=2, keepdims=True))

        evrk = rk_all * ev[:, None, :]
        evwv = ev * wv
        kmat = jnp.concatenate(
            [wk[:, None, :], rk_all, evrk, wv[:, None, :], evwv[:, None, :]],
            axis=1,
        )
        qmat = jnp.concatenate(
            [jnp.ones((B, 1, MEM), jnp.float32), ev[:, None, :], (ev * ev)[:, None, :]],
            axis=1,
        )
        sims = _bdot(kmat, m, 2, 1)
        sq = _bdot(qmat, msq, 2, 1)
        s_rv = jnp.sum(rk_all * wv[:, None, :], axis=2, keepdims=True)
        s_vv = jnp.sum(wv * wv, axis=1)[:, None, None]

        simw = sims[:, 0] / (nrm[...] + 1e-6) * (wbeta / (wknorm + 1e-6))
        e = jnp.exp(jnp.maximum(simw - wbeta, -60.0))
        ww = e / jnp.sum(e, axis=-1, keepdims=True)

        ww3 = ww[:, None, :]
        simr_raw = sims[:, 1 : 1 + NHEAD] - ww3 * sims[:, 1 + NHEAD : 1 + 2 * NHEAD] + ww3 * s_rv
        nrm2_new = (
            sq[:, 0:1]
            - 2.0 * ww3 * sq[:, 1:2]
            + (ww3 * ww3) * sq[:, 2:3]
            + 2.0 * ww3 * sims[:, 9:10]
            - 2.0 * (ww3 * ww3) * sims[:, 10:11]
            + (ww3 * ww3) * s_vv
        )
        nrm_new = jnp.sqrt(jnp.maximum(nrm2_new, 0.0))
        nrm[...] = nrm_new[:, 0]

        simr = simr_raw / (nrm_new + 1e-6) * (rbeta[:, :, None] / (rknorm + 1e-6))
        er = jnp.exp(jnp.maximum(simr - rbeta[:, :, None], -60.0))
        wr = er / jnp.sum(er, axis=-1, keepdims=True)

        wrw = wr * ww3
        wr2 = jnp.concatenate([wr, wrw], axis=1)
        rsum = _bdot(wr2, m, 2, 2)
        srw = jnp.sum(wrw, axis=2, keepdims=True)
        reads = (
            rsum[:, 0:NHEAD]
            - rsum[:, NHEAD : 2 * NHEAD] * ev[:, None, :]
            + srw * wv[:, None, :]
        )
        r_new = reads.reshape(B, NHEAD * MEM)
        rB[...] = r_new
        outs_r[t] = r_new
        wwS[...] = ww
        evS[...] = ev
        wvS[...] = wv
        return carry

    lax.fori_loop(0, T, step, 0)

    wouth = wouth_ref[...]
    woutr = woutr_ref[...]
    wfc = wfc_ref[...]
    bfc = bfc_ref[...]
    for b in range(B):
        src_b = (
            _dot(outs_h[:, b, :], wouth, 1, 1)
            + _dot(outs_r[:, b, :], woutr, 1, 1)
        )
        out_ref[b] = _dot(wfc, src_b, 1, 1) + bfc


def _recurrence(xs, wall, bl, wifp, bifp, wouth, woutr, wfc, bfc):
    return pl.pallas_call(
        _dnc_body,
        out_shape=jax.ShapeDtypeStruct((B, VOCAB, T), jnp.float32),
        scratch_shapes=[
            pltpu.VMEM((B, MEM, NCELLS), jnp.float32),
            pltpu.VMEM((B, MODEL), jnp.float32),
            pltpu.VMEM((B, MODEL), jnp.float32),
            pltpu.VMEM((B, NHEAD * MEM), jnp.float32),
            pltpu.VMEM((B, NCELLS), jnp.float32),
            pltpu.VMEM((T, B, MODEL), jnp.float32),
            pltpu.VMEM((T, B, NHEAD * MEM), jnp.float32),
            pltpu.VMEM((B, NCELLS), jnp.float32),
            pltpu.VMEM((B, MEM), jnp.float32),
            pltpu.VMEM((B, MEM), jnp.float32),
        ],
    )(xs, wall, bl, wifp, bifp, wouth, woutr, wfc, bfc)


def kernel(input, emb, W_ih, W_hh, b_lstm, W_if, b_if, W_out, b_out, W_fc, b_fc):
    idx = jnp.swapaxes(input, 0, 1).reshape(T * B).astype(jnp.int32)
    rows = _make_sc_gather()(emb, idx)
    xs = rows.reshape(T, B, MODEL)

    wall = jnp.concatenate([W_ih, W_hh], axis=1)
    bl = b_lstm.reshape(1, -1)
    iface_dim = W_if.shape[0]
    wifp = jnp.zeros((IFACE_PAD, MODEL), jnp.float32).at[:iface_dim].set(W_if)
    bifp = jnp.zeros((1, IFACE_PAD), jnp.float32).at[0, :iface_dim].set(b_if)
    wouth = W_out[:, :MODEL]
    woutr = W_out[:, MODEL:]
    bfc = (W_fc @ b_out + b_fc).reshape(-1, 1)

    return _recurrence(xs, wall, bl, wifp, bifp, wouth, woutr, W_fc, bfc)


# speedup vs baseline: 1.1384x; 1.0138x over previous
import functools

import jax
import jax.numpy as jnp
from jax import lax
from jax.experimental import pallas as pl
from jax.experimental.pallas import tpu as pltpu
from jax.experimental.pallas import tpu_sc as plsc

MODEL = 128
NHEAD = 4
NCELLS = 512
VOCAB = 1000
MEM = 64
B, T = 8, 128
IFACE_PAD = 512


def _make_sc_gather():
    info = plsc.get_sparse_core_info()
    nc, ns = info.num_cores, info.num_subcores
    nw = nc * ns
    n_idx = T * B
    per_w = n_idx // nw
    mesh = plsc.VectorSubcoreMesh(core_axis_name="c", subcore_axis_name="s")

    @functools.partial(
        pl.kernel,
        mesh=mesh,
        out_type=jax.ShapeDtypeStruct((n_idx, MODEL), jnp.float32),
        scratch_types=[
            pltpu.VMEM((per_w,), jnp.int32),
            pltpu.VMEM((per_w, MODEL), jnp.float32),
            pltpu.SemaphoreType.DMA,
        ],
    )
    def gather(table_hbm, idx_hbm, out_hbm, idx_v, rows_v, sem):
        wid = lax.axis_index("s") * nc + lax.axis_index("c")
        base = wid * per_w
        pltpu.sync_copy(idx_hbm.at[pl.ds(base, per_w)], idx_v)
        pltpu.async_copy(table_hbm.at[idx_v], rows_v, sem).wait()
        pltpu.sync_copy(rows_v, out_hbm.at[pl.ds(base, per_w)])

    return gather


def _dot(a, b, ca, cb):
    return lax.dot_general(
        a, b, (((ca,), (cb,)), ((), ())), preferred_element_type=jnp.float32
    )


def _bdot(a, b, ca, cb):
    return lax.dot_general(
        a, b, (((ca,), (cb,)), ((0,), (0,))), preferred_element_type=jnp.float32
    )


def _softplus(x):
    return jnp.maximum(x, 0.0) + jnp.log(1.0 + jnp.exp(-jnp.abs(x)))


def _dnc_body(
    xs_ref, wallx_ref, wallrh_ref, bl_ref, wif_ref, bif_ref,
    wouth_ref, woutr_ref, wfc_ref, bfc_ref, out_ref,
    m8, hB, cB, rB, nrm, outs_h, outs_r, wwS, evS, wvS, xg,
):
    m8[...] = jnp.zeros_like(m8)
    hB[...] = jnp.zeros_like(hB)
    cB[...] = jnp.zeros_like(cB)
    rB[...] = jnp.zeros_like(rB)
    nrm[...] = jnp.zeros_like(nrm)
    wwS[...] = jnp.zeros_like(wwS)
    evS[...] = jnp.zeros_like(evS)
    wvS[...] = jnp.zeros_like(wvS)

    xflat = xs_ref[...].reshape(T * B, MODEL)
    xg[...] = _dot(xflat, wallx_ref[...], 1, 1).reshape(T, B, 4 * MODEL)

    wallrh = wallrh_ref[...]
    bl = bl_ref[...]
    wif = wif_ref[...]
    bif = bif_ref[...]

    def step(t, carry):
        wwp = wwS[...][:, None, :]
        m = m8[...] * (1.0 - wwp * evS[...][:, :, None]) + wwp * wvS[...][:, :, None]
        m8[...] = m
        msq = m * m

        catv = jnp.concatenate([rB[...], hB[...]], axis=1)
        gates = xg[t] + _dot(catv, wallrh, 1, 1) + bl
        ig = jax.nn.sigmoid(gates[:, 0:MODEL])
        fg = jax.nn.sigmoid(gates[:, MODEL : 2 * MODEL])
        gg = jnp.tanh(gates[:, 2 * MODEL : 3 * MODEL])
        og = jax.nn.sigmoid(gates[:, 3 * MODEL : 4 * MODEL])
        c_new = fg * cB[...] + ig * gg
        h_new = og * jnp.tanh(c_new)
        cB[...] = c_new
        hB[...] = h_new
        outs_h[t] = h_new

        iface = _dot(h_new, wif, 1, 1) + bif
        wk = iface[:, NHEAD * MEM : NHEAD * MEM + MEM]
        wv = iface[:, NHEAD * MEM + MEM : NHEAD * MEM + 2 * MEM]
        ev = jax.nn.sigmoid(iface[:, NHEAD * MEM + 2 * MEM : NHEAD * MEM + 3 * MEM])
        betas = _softplus(iface[:, NHEAD * MEM + 3 * MEM : NHEAD * MEM + 3 * MEM + NHEAD + 1]) + 1.0
        rbeta = betas[:, 0:NHEAD]
        wbeta = betas[:, NHEAD : NHEAD + 1]

        rk_all = iface[:, 0 : NHEAD * MEM].reshape(B, NHEAD, MEM)
        wknorm = jnp.sqrt(jnp.sum(wk * wk, axis=1, keepdims=True))
        rknorm = jnp.sqrt(jnp.sum(rk_all * rk_all, axis=2, keepdims=True))

        evrk = rk_all * ev[:, None, :]
        evwv = ev * wv
        kmat = jnp.concatenate(
            [wk[:, None, :], rk_all, evrk, wv[:, None, :], evwv[:, None, :]],
            axis=1,
        )
        qmat = jnp.concatenate(
            [jnp.ones((B, 1, MEM), jnp.float32), ev[:, None, :], (ev * ev)[:, None, :]],
            axis=1,
        )
        sims = _bdot(kmat, m, 2, 1)
        sq = _bdot(qmat, msq, 2, 1)
        s_rv = jnp.sum(rk_all * wv[:, None, :], axis=2, keepdims=True)
        s_vv = jnp.sum(wv * wv, axis=1)[:, None, None]

        simw = sims[:, 0] / (nrm[...] + 1e-6) * (wbeta / (wknorm + 1e-6))
        e = jnp.exp(jnp.maximum(simw - wbeta, -60.0))
        ww = e / jnp.sum(e, axis=-1, keepdims=True)

        ww3 = ww[:, None, :]
        simr_raw = sims[:, 1 : 1 + NHEAD] - ww3 * sims[:, 1 + NHEAD : 1 + 2 * NHEAD] + ww3 * s_rv
        nrm2_new = (
            sq[:, 0:1]
            - 2.0 * ww3 * sq[:, 1:2]
            + (ww3 * ww3) * sq[:, 2:3]
            + 2.0 * ww3 * sims[:, 9:10]
            - 2.0 * (ww3 * ww3) * sims[:, 10:11]
            + (ww3 * ww3) * s_vv
        )
        nrm_new = jnp.sqrt(jnp.maximum(nrm2_new, 0.0))
        nrm[...] = nrm_new[:, 0]

        simr = simr_raw / (nrm_new + 1e-6) * (rbeta[:, :, None] / (rknorm + 1e-6))
        er = jnp.exp(jnp.maximum(simr - rbeta[:, :, None], -60.0))
        wr = er / jnp.sum(er, axis=-1, keepdims=True)

        wrw = wr * ww3
        wr2 = jnp.concatenate([wr, wrw], axis=1)
        rsum = _bdot(wr2, m, 2, 2)
        srw = jnp.sum(wrw, axis=2, keepdims=True)
        reads = (
            rsum[:, 0:NHEAD]
            - rsum[:, NHEAD : 2 * NHEAD] * ev[:, None, :]
            + srw * wv[:, None, :]
        )
        r_new = reads.reshape(B, NHEAD * MEM)
        rB[...] = r_new
        outs_r[t] = r_new
        wwS[...] = ww
        evS[...] = ev
        wvS[...] = wv
        return carry

    lax.fori_loop(0, T, step, 0)

    wouth = wouth_ref[...]
    woutr = woutr_ref[...]
    wfc = wfc_ref[...]
    bfc = bfc_ref[...]
    for b in range(B):
        src_b = (
            _dot(outs_h[:, b, :], wouth, 1, 1)
            + _dot(outs_r[:, b, :], woutr, 1, 1)
        )
        out_ref[b] = _dot(wfc, src_b, 1, 1) + bfc


def _recurrence(xs, wallx, wallrh, bl, wifp, bifp, wouth, woutr, wfc, bfc):
    return pl.pallas_call(
        _dnc_body,
        out_shape=jax.ShapeDtypeStruct((B, VOCAB, T), jnp.float32),
        scratch_shapes=[
            pltpu.VMEM((B, MEM, NCELLS), jnp.float32),
            pltpu.VMEM((B, MODEL), jnp.float32),
            pltpu.VMEM((B, MODEL), jnp.float32),
            pltpu.VMEM((B, NHEAD * MEM), jnp.float32),
            pltpu.VMEM((B, NCELLS), jnp.float32),
            pltpu.VMEM((T, B, MODEL), jnp.float32),
            pltpu.VMEM((T, B, NHEAD * MEM), jnp.float32),
            pltpu.VMEM((B, NCELLS), jnp.float32),
            pltpu.VMEM((B, MEM), jnp.float32),
            pltpu.VMEM((B, MEM), jnp.float32),
            pltpu.VMEM((T, B, 4 * MODEL), jnp.float32),
        ],
    )(xs, wallx, wallrh, bl, wifp, bifp, wouth, woutr, wfc, bfc)


def kernel(input, emb, W_ih, W_hh, b_lstm, W_if, b_if, W_out, b_out, W_fc, b_fc):
    idx = jnp.swapaxes(input, 0, 1).reshape(T * B).astype(jnp.int32)
    rows = _make_sc_gather()(emb, idx)
    xs = rows.reshape(T, B, MODEL)

    wallx = W_ih[:, :MODEL]
    wallrh = jnp.concatenate([W_ih[:, MODEL:], W_hh], axis=1)
    bl = b_lstm.reshape(1, -1)
    iface_dim = W_if.shape[0]
    wifp = jnp.zeros((IFACE_PAD, MODEL), jnp.float32).at[:iface_dim].set(W_if)
    bifp = jnp.zeros((1, IFACE_PAD), jnp.float32).at[0, :iface_dim].set(b_if)
    wouth = W_out[:, :MODEL]
    woutr = W_out[:, MODEL:]
    bfc = (W_fc @ b_out + b_fc).reshape(-1, 1)

    return _recurrence(xs, wallx, wallrh, bl, wifp, bifp, wouth, woutr, W_fc, bfc)


# speedup vs baseline: 1.1386x; 1.0001x over previous
import functools

import jax
import jax.numpy as jnp
from jax import lax
from jax.experimental import pallas as pl
from jax.experimental.pallas import tpu as pltpu
from jax.experimental.pallas import tpu_sc as plsc

MODEL = 128
NHEAD = 4
NCELLS = 512
VOCAB = 1000
MEM = 64
B, T = 8, 128
IFACE_PAD = 512


def _make_sc_gather():
    info = plsc.get_sparse_core_info()
    nc, ns = info.num_cores, info.num_subcores
    nw = nc * ns
    n_idx = T * B
    per_w = n_idx // nw
    mesh = plsc.VectorSubcoreMesh(core_axis_name="c", subcore_axis_name="s")

    @functools.partial(
        pl.kernel,
        mesh=mesh,
        out_type=jax.ShapeDtypeStruct((n_idx, MODEL), jnp.float32),
        scratch_types=[
            pltpu.VMEM((per_w,), jnp.int32),
            pltpu.VMEM((per_w, MODEL), jnp.float32),
            pltpu.SemaphoreType.DMA,
        ],
    )
    def gather(table_hbm, idx_hbm, out_hbm, idx_v, rows_v, sem):
        wid = lax.axis_index("s") * nc + lax.axis_index("c")
        base = wid * per_w
        pltpu.sync_copy(idx_hbm.at[pl.ds(base, per_w)], idx_v)
        pltpu.async_copy(table_hbm.at[idx_v], rows_v, sem).wait()
        pltpu.sync_copy(rows_v, out_hbm.at[pl.ds(base, per_w)])

    return gather


def _dot(a, b, ca, cb):
    return lax.dot_general(
        a, b, (((ca,), (cb,)), ((), ())), preferred_element_type=jnp.float32
    )


def _bdot(a, b, ca, cb):
    return lax.dot_general(
        a, b, (((ca,), (cb,)), ((0,), (0,))), preferred_element_type=jnp.float32
    )


def _softplus(x):
    return jnp.maximum(x, 0.0) + jnp.log(1.0 + jnp.exp(-jnp.abs(x)))


def _dnc_body(
    xs_ref, wallx_ref, wallrh_ref, bl_ref, wif_ref, bif_ref,
    wouth_ref, woutr_ref, wfc_ref, bfc_ref, out_ref,
    m8, catrh, cB, nrm, outs_h, outs_r, wwS, evS, wvS, xg,
):
    m8[...] = jnp.zeros_like(m8)
    catrh[...] = jnp.zeros_like(catrh)
    cB[...] = jnp.zeros_like(cB)
    nrm[...] = jnp.zeros_like(nrm)
    wwS[...] = jnp.zeros_like(wwS)
    evS[...] = jnp.zeros_like(evS)
    wvS[...] = jnp.zeros_like(wvS)

    xflat = xs_ref[...].reshape(T * B, MODEL)
    xg[...] = _dot(xflat, wallx_ref[...], 1, 1).reshape(T, B, 4 * MODEL)

    wallrh = wallrh_ref[...]
    bl = bl_ref[...]
    wif = wif_ref[...]
    bif = bif_ref[...]

    def step(t, carry):
        wwp = wwS[...][:, None, :]
        m = m8[...] * (1.0 - wwp * evS[...][:, :, None]) + wwp * wvS[...][:, :, None]
        m8[...] = m
        msq = m * m

        gates = xg[t] + _dot(catrh[...], wallrh, 1, 1) + bl
        ig = jax.nn.sigmoid(gates[:, 0:MODEL])
        fg = jax.nn.sigmoid(gates[:, MODEL : 2 * MODEL])
        gg = jnp.tanh(gates[:, 2 * MODEL : 3 * MODEL])
        og = jax.nn.sigmoid(gates[:, 3 * MODEL : 4 * MODEL])
        c_new = fg * cB[...] + ig * gg
        h_new = og * jnp.tanh(c_new)
        cB[...] = c_new
        catrh[:, NHEAD * MEM :] = h_new
        outs_h[t] = h_new

        iface = _dot(h_new, wif, 1, 1) + bif
        wk = iface[:, NHEAD * MEM : NHEAD * MEM + MEM]
        wv = iface[:, NHEAD * MEM + MEM : NHEAD * MEM + 2 * MEM]
        ev = jax.nn.sigmoid(iface[:, NHEAD * MEM + 2 * MEM : NHEAD * MEM + 3 * MEM])
        betas = _softplus(iface[:, NHEAD * MEM + 3 * MEM : NHEAD * MEM + 3 * MEM + NHEAD + 1]) + 1.0
        rbeta = betas[:, 0:NHEAD]
        wbeta = betas[:, NHEAD : NHEAD + 1]

        rk_all = iface[:, 0 : NHEAD * MEM].reshape(B, NHEAD, MEM)
        wknorm = jnp.sqrt(jnp.sum(wk * wk, axis=1, keepdims=True))
        rknorm = jnp.sqrt(jnp.sum(rk_all * rk_all, axis=2, keepdims=True))

        evrk = rk_all * ev[:, None, :]
        evwv = ev * wv
        kmat = jnp.concatenate(
            [wk[:, None, :], rk_all, evrk, wv[:, None, :], evwv[:, None, :]],
            axis=1,
        )
        qmat = jnp.concatenate(
            [jnp.ones((B, 1, MEM), jnp.float32), ev[:, None, :], (ev * ev)[:, None, :]],
            axis=1,
        )
        sims = _bdot(kmat, m, 2, 1)
        sq = _bdot(qmat, msq, 2, 1)
        s_rv = jnp.sum(rk_all * wv[:, None, :], axis=2, keepdims=True)
        s_vv = jnp.sum(wv * wv, axis=1)[:, None, None]

        simw = sims[:, 0] / (nrm[...] + 1e-6) * (wbeta / (wknorm + 1e-6))
        e = jnp.exp(jnp.maximum(simw - wbeta, -60.0))
        ww = e / jnp.sum(e, axis=-1, keepdims=True)

        ww3 = ww[:, None, :]
        simr_raw = sims[:, 1 : 1 + NHEAD] - ww3 * sims[:, 1 + NHEAD : 1 + 2 * NHEAD] + ww3 * s_rv
        nrm2_new = (
            sq[:, 0:1]
            - 2.0 * ww3 * sq[:, 1:2]
            + (ww3 * ww3) * sq[:, 2:3]
            + 2.0 * ww3 * sims[:, 9:10]
            - 2.0 * (ww3 * ww3) * sims[:, 10:11]
            + (ww3 * ww3) * s_vv
        )
        nrm_new = jnp.sqrt(jnp.maximum(nrm2_new, 0.0))
        nrm[...] = nrm_new[:, 0]

        simr = simr_raw / (nrm_new + 1e-6) * (rbeta[:, :, None] / (rknorm + 1e-6))
        er = jnp.exp(jnp.maximum(simr - rbeta[:, :, None], -60.0))
        wr = er / jnp.sum(er, axis=-1, keepdims=True)

        wrw = wr * ww3
        wr2 = jnp.concatenate([wr, wrw], axis=1)
        rsum = _bdot(wr2, m, 2, 2)
        srw = jnp.sum(wrw, axis=2, keepdims=True)
        reads = (
            rsum[:, 0:NHEAD]
            - rsum[:, NHEAD : 2 * NHEAD] * ev[:, None, :]
            + srw * wv[:, None, :]
        )
        r_new = reads.reshape(B, NHEAD * MEM)
        catrh[:, 0 : NHEAD * MEM] = r_new
        outs_r[t] = r_new
        wwS[...] = ww
        evS[...] = ev
        wvS[...] = wv
        return carry

    lax.fori_loop(0, T, step, 0)

    wouth = wouth_ref[...]
    woutr = woutr_ref[...]
    wfc = wfc_ref[...]
    bfc = bfc_ref[...]
    for b in range(B):
        src_b = (
            _dot(outs_h[:, b, :], wouth, 1, 1)
            + _dot(outs_r[:, b, :], woutr, 1, 1)
        )
        out_ref[b] = _dot(wfc, src_b, 1, 1) + bfc


def _recurrence(xs, wallx, wallrh, bl, wifp, bifp, wouth, woutr, wfc, bfc):
    return pl.pallas_call(
        _dnc_body,
        out_shape=jax.ShapeDtypeStruct((B, VOCAB, T), jnp.float32),
        scratch_shapes=[
            pltpu.VMEM((B, MEM, NCELLS), jnp.float32),
            pltpu.VMEM((B, NHEAD * MEM + MODEL), jnp.float32),
            pltpu.VMEM((B, MODEL), jnp.float32),
            pltpu.VMEM((B, NCELLS), jnp.float32),
            pltpu.VMEM((T, B, MODEL), jnp.float32),
            pltpu.VMEM((T, B, NHEAD * MEM), jnp.float32),
            pltpu.VMEM((B, NCELLS), jnp.float32),
            pltpu.VMEM((B, MEM), jnp.float32),
            pltpu.VMEM((B, MEM), jnp.float32),
            pltpu.VMEM((T, B, 4 * MODEL), jnp.float32),
        ],
    )(xs, wallx, wallrh, bl, wifp, bifp, wouth, woutr, wfc, bfc)


def kernel(input, emb, W_ih, W_hh, b_lstm, W_if, b_if, W_out, b_out, W_fc, b_fc):
    idx = jnp.swapaxes(input, 0, 1).reshape(T * B).astype(jnp.int32)
    rows = _make_sc_gather()(emb, idx)
    xs = rows.reshape(T, B, MODEL)

    wallx = W_ih[:, :MODEL]
    wallrh = jnp.concatenate([W_ih[:, MODEL:], W_hh], axis=1)
    bl = b_lstm.reshape(1, -1)
    iface_dim = W_if.shape[0]
    wifp = jnp.zeros((IFACE_PAD, MODEL), jnp.float32).at[:iface_dim].set(W_if)
    bifp = jnp.zeros((1, IFACE_PAD), jnp.float32).at[0, :iface_dim].set(b_if)
    wouth = W_out[:, :MODEL]
    woutr = W_out[:, MODEL:]
    bfc = (W_fc @ b_out + b_fc).reshape(-1, 1)

    return _recurrence(xs, wallx, wallrh, bl, wifp, bifp, wouth, woutr, W_fc, bfc)
